# baseline (speedup 1.0000x reference)
import jax
import jax.numpy as jnp
from jax import lax
from jax.experimental import pallas as pl
from jax.experimental.pallas import tpu as pltpu

_SLOPE = 5.0
_BUDGET = 1.0 / 16.0 - 1.0 / 128.0
_RATIO = 128
_C_LO = 146
_C_HI = 174
_HW = 102400
_ROWS = 800
_LANES = 128

_KS0 = 0
_KS1 = 42
_KS2 = 0x1BD11BDA ^ _KS0 ^ _KS1


def _rotl(x, d):
    return (x << jnp.uint32(d)) | (x >> jnp.uint32(32 - d))


def _threefry2x32(x0, x1):
    ks0 = jnp.uint32(_KS0)
    ks1 = jnp.uint32(_KS1)
    ks2 = jnp.uint32(_KS2)
    x0 = x0 + ks0
    x1 = x1 + ks1
    r_a = (13, 15, 26, 6)
    r_b = (17, 29, 16, 24)

    def four_rounds(x0, x1, rots):
        for r in rots:
            x0 = x0 + x1
            x1 = _rotl(x1, r) ^ x0
        return x0, x1

    x0, x1 = four_rounds(x0, x1, r_a)
    x0 = x0 + ks1
    x1 = x1 + jnp.uint32(_KS2 + 1)
    x0, x1 = four_rounds(x0, x1, r_b)
    x0 = x0 + ks2
    x1 = x1 + jnp.uint32(_KS0 + 2)
    x0, x1 = four_rounds(x0, x1, r_a)
    x0 = x0 + ks0
    x1 = x1 + jnp.uint32(_KS1 + 3)
    x0, x1 = four_rounds(x0, x1, r_b)
    x0 = x0 + ks1
    x1 = x1 + jnp.uint32(_KS2 + 4)
    x0, x1 = four_rounds(x0, x1, r_a)
    x0 = x0 + ks2
    x1 = x1 + jnp.uint32(_KS0 + 5)
    return x0, x1


def _bits_to_uniform(bits):
    fb = (bits >> jnp.uint32(9)) | jnp.uint32(0x3F800000)
    return lax.bitcast_convert_type(fb, jnp.float32) - jnp.float32(1.0)


def _body(w_ref, ks_ref, oks_ref, mask_ref, mr_ref):
    b = pl.program_id(0)

    @pl.when(b == 0)
    def _prep():
        prob = jax.nn.sigmoid(jnp.float32(_SLOPE) * w_ref[...])
        j = (
            lax.broadcasted_iota(jnp.int32, (_ROWS, _LANES), 0) * _LANES
            + lax.broadcasted_iota(jnp.int32, (_ROWS, _LANES), 1)
        )
        h = j // 320
        w = j - h * 320
        inside = (h >= _C_LO) & (h < _C_HI) & (w >= _C_LO) & (w < _C_HI)
        prob = jnp.where(inside, jnp.float32(0.0), prob)
        xbar = jnp.mean(prob)
        r = jnp.float32(_BUDGET) / xbar
        beta = (jnp.float32(1.0) - jnp.float32(_BUDGET)) / (jnp.float32(1.0) - xbar)
        mr_ref[...] = jnp.where(
            r <= jnp.float32(1.0),
            prob * r,
            jnp.float32(1.0) - (jnp.float32(1.0) - prob) * beta,
        )

    mr = mr_ref[...]
    ju = (
        lax.broadcasted_iota(jnp.uint32, (_ROWS, _LANES), 0) * jnp.uint32(_LANES)
        + lax.broadcasted_iota(jnp.uint32, (_ROWS, _LANES), 1)
    )
    x1 = ju + lax.convert_element_type(b * _HW, jnp.uint32)
    x0 = jnp.zeros((_ROWS, _LANES), jnp.uint32)
    o0, o1 = _threefry2x32(x0, x1)
    m = (mr > _bits_to_uniform(o0 ^ o1)).astype(jnp.float32)
    mask_ref[0] = m
    oks_ref[0] = ks_ref[0] * m[None]


@jax.jit
def kernel(kspace, weight):
    B, C = kspace.shape[0], kspace.shape[1]
    ks = kspace.reshape(B, C, _ROWS, _LANES)
    w = weight.reshape(_ROWS, _LANES)
    oks, mask = pl.pallas_call(
        _body,
        grid=(B,),
        in_specs=[
            pl.BlockSpec((_ROWS, _LANES), lambda b: (0, 0)),
            pl.BlockSpec((1, C, _ROWS, _LANES), lambda b: (b, 0, 0, 0)),
        ],
        out_specs=[
            pl.BlockSpec((1, C, _ROWS, _LANES), lambda b: (b, 0, 0, 0)),
            pl.BlockSpec((1, _ROWS, _LANES), lambda b: (b, 0, 0)),
        ],
        out_shape=[
            jax.ShapeDtypeStruct((B, C, _ROWS, _LANES), jnp.float32),
            jax.ShapeDtypeStruct((B, _ROWS, _LANES), jnp.float32),
        ],
        scratch_shapes=[pltpu.VMEM((_ROWS, _LANES), jnp.float32)],
    )(w, ks)
    return (
        oks.reshape(B, C, 320, 320),
        mask.reshape(B, 320, 320),
        jnp.asarray(_RATIO, dtype=jnp.int32),
    )


# speedup vs baseline: 2.1246x; 2.1246x over previous
import jax
import jax.numpy as jnp
from jax import lax
from jax.experimental import pallas as pl
from jax.experimental.pallas import tpu as pltpu

_SLOPE = 5.0
_BUDGET = 1.0 / 16.0 - 1.0 / 128.0
_RATIO = 128
_C_LO = 146
_C_HI = 174
_H = 320
_W = 320
_HW = _H * _W

_KS0 = 0
_KS1 = 42
_KS2 = 0x1BD11BDA ^ _KS0 ^ _KS1


def _rotl(x, d):
    return (x << jnp.uint32(d)) | (x >> jnp.uint32(32 - d))


def _threefry2x32(x0, x1):
    ks0 = jnp.uint32(_KS0)
    ks1 = jnp.uint32(_KS1)
    ks2 = jnp.uint32(_KS2)
    x0 = x0 + ks0
    x1 = x1 + ks1
    r_a = (13, 15, 26, 6)
    r_b = (17, 29, 16, 24)

    def four_rounds(x0, x1, rots):
        for r in rots:
            x0 = x0 + x1
            x1 = _rotl(x1, r) ^ x0
        return x0, x1

    x0, x1 = four_rounds(x0, x1, r_a)
    x0 = x0 + ks1
    x1 = x1 + jnp.uint32(_KS2 + 1)
    x0, x1 = four_rounds(x0, x1, r_b)
    x0 = x0 + ks2
    x1 = x1 + jnp.uint32(_KS0 + 2)
    x0, x1 = four_rounds(x0, x1, r_a)
    x0 = x0 + ks0
    x1 = x1 + jnp.uint32(_KS1 + 3)
    x0, x1 = four_rounds(x0, x1, r_b)
    x0 = x0 + ks1
    x1 = x1 + jnp.uint32(_KS2 + 4)
    x0, x1 = four_rounds(x0, x1, r_a)
    x0 = x0 + ks2
    x1 = x1 + jnp.uint32(_KS0 + 5)
    return x0, x1


def _bits_to_uniform(bits):
    fb = (bits >> jnp.uint32(9)) | jnp.uint32(0x3F800000)
    return lax.bitcast_convert_type(fb, jnp.float32) - jnp.float32(1.0)


def _body(w_ref, ks_ref, oks_ref, mask_ref, mr_ref):
    b = pl.program_id(0)
    row = lax.broadcasted_iota(jnp.int32, (_H, _W), 0)
    col = lax.broadcasted_iota(jnp.int32, (_H, _W), 1)

    @pl.when(b == 0)
    def _prep():
        prob = jax.nn.sigmoid(jnp.float32(_SLOPE) * w_ref[...])
        inside = (row >= _C_LO) & (row < _C_HI) & (col >= _C_LO) & (col < _C_HI)
        prob = jnp.where(inside, jnp.float32(0.0), prob)
        xbar = jnp.mean(prob)
        r = jnp.float32(_BUDGET) / xbar
        beta = (jnp.float32(1.0) - jnp.float32(_BUDGET)) / (jnp.float32(1.0) - xbar)
        mr_ref[...] = jnp.where(
            r <= jnp.float32(1.0),
            prob * r,
            jnp.float32(1.0) - (jnp.float32(1.0) - prob) * beta,
        )

    mr = mr_ref[...]
    ju = (row * _W + col).astype(jnp.uint32)
    x1 = ju + lax.convert_element_type(b * _HW, jnp.uint32)
    x0 = jnp.zeros((_H, _W), jnp.uint32)
    o0, o1 = _threefry2x32(x0, x1)
    m = (mr > _bits_to_uniform(o0 ^ o1)).astype(jnp.float32)
    mask_ref[0] = m
    oks_ref[0] = ks_ref[0] * m[None]


@jax.jit
def kernel(kspace, weight):
    B, C = kspace.shape[0], kspace.shape[1]
    oks, mask = pl.pallas_call(
        _body,
        grid=(B,),
        in_specs=[
            pl.BlockSpec((_H, _W), lambda b: (0, 0)),
            pl.BlockSpec((1, C, _H, _W), lambda b: (b, 0, 0, 0)),
        ],
        out_specs=[
            pl.BlockSpec((1, C, _H, _W), lambda b: (b, 0, 0, 0)),
            pl.BlockSpec((1, _H, _W), lambda b: (b, 0, 0)),
        ],
        out_shape=[
            jax.ShapeDtypeStruct((B, C, _H, _W), jnp.float32),
            jax.ShapeDtypeStruct((B, _H, _W), jnp.float32),
        ],
        scratch_shapes=[pltpu.VMEM((_H, _W), jnp.float32)],
    )(weight, kspace)
    return (
        oks,
        mask,
        jnp.asarray(_RATIO, dtype=jnp.int32),
    )


# speedup vs baseline: 2.2041x; 1.0374x over previous
import jax
import jax.numpy as jnp
from jax import lax
from jax.experimental import pallas as pl
from jax.experimental.pallas import tpu as pltpu

_SLOPE = 5.0
_BUDGET = 1.0 / 16.0 - 1.0 / 128.0
_RATIO = 128
_C_LO = 146
_C_HI = 174
_H = 320
_W = 320
_HW = _H * _W

_KS0 = 0
_KS1 = 42
_KS2 = 0x1BD11BDA ^ _KS0 ^ _KS1


def _rotl(x, d):
    return (x << jnp.uint32(d)) | (x >> jnp.uint32(32 - d))


def _threefry2x32_zero_x0(x1_plus_ks1):
    ks0 = jnp.uint32(_KS0)
    ks1 = jnp.uint32(_KS1)
    ks2 = jnp.uint32(_KS2)
    r_a = (13, 15, 26, 6)
    r_b = (17, 29, 16, 24)

    def four_rounds(x0, x1, rots):
        for r in rots:
            x0 = x0 + x1
            x1 = _rotl(x1, r) ^ x0
        return x0, x1

    x0 = x1_plus_ks1
    x1 = _rotl(x1_plus_ks1, 13) ^ x0
    x0, x1 = four_rounds(x0, x1, (15, 26, 6))
    x0 = x0 + ks1
    x1 = x1 + jnp.uint32(_KS2 + 1)
    x0, x1 = four_rounds(x0, x1, r_b)
    x0 = x0 + ks2
    x1 = x1 + jnp.uint32(_KS0 + 2)
    x0, x1 = four_rounds(x0, x1, r_a)
    x0 = x0 + ks0
    x1 = x1 + jnp.uint32(_KS1 + 3)
    x0, x1 = four_rounds(x0, x1, r_b)
    x0 = x0 + ks1
    x1 = x1 + jnp.uint32(_KS2 + 4)
    x0, x1 = four_rounds(x0, x1, r_a)
    x0 = x0 + ks2
    x1 = x1 + jnp.uint32(_KS0 + 5)
    return x0, x1


def _body(w_ref, ks_ref, oks_ref, mask_ref, thr_ref, ju_ref):
    b = pl.program_id(0)

    @pl.when(b == 0)
    def _prep():
        row = lax.broadcasted_iota(jnp.int32, (_H, _W), 0)
        col = lax.broadcasted_iota(jnp.int32, (_H, _W), 1)
        prob = jax.nn.sigmoid(jnp.float32(_SLOPE) * w_ref[...])
        inside = (row >= _C_LO) & (row < _C_HI) & (col >= _C_LO) & (col < _C_HI)
        prob = jnp.where(inside, jnp.float32(0.0), prob)
        xbar = jnp.mean(prob)
        r = jnp.float32(_BUDGET) / xbar
        beta = (jnp.float32(1.0) - jnp.float32(_BUDGET)) / (jnp.float32(1.0) - xbar)
        mr = jnp.where(
            r <= jnp.float32(1.0),
            prob * r,
            jnp.float32(1.0) - (jnp.float32(1.0) - prob) * beta,
        )
        thr_ref[...] = jnp.ceil(mr * jnp.float32(8388608.0)).astype(jnp.int32)
        ju_ref[...] = (row * _W + col).astype(jnp.uint32) + jnp.uint32(_KS1)

    x1 = ju_ref[...] + lax.convert_element_type(b * _HW, jnp.uint32)
    o0, o1 = _threefry2x32_zero_x0(x1)
    mant = lax.convert_element_type((o0 ^ o1) >> jnp.uint32(9), jnp.int32)
    m = (mant < thr_ref[...]).astype(jnp.float32)
    mask_ref[0] = m
    oks_ref[0] = ks_ref[0] * m[None]


@jax.jit
def kernel(kspace, weight):
    B, C = kspace.shape[0], kspace.shape[1]
    oks, mask = pl.pallas_call(
        _body,
        grid=(B,),
        in_specs=[
            pl.BlockSpec((_H, _W), lambda b: (0, 0)),
            pl.BlockSpec((1, C, _H, _W), lambda b: (b, 0, 0, 0)),
        ],
        out_specs=[
            pl.BlockSpec((1, C, _H, _W), lambda b: (b, 0, 0, 0)),
            pl.BlockSpec((1, _H, _W), lambda b: (b, 0, 0)),
        ],
        out_shape=[
            jax.ShapeDtypeStruct((B, C, _H, _W), jnp.float32),
            jax.ShapeDtypeStruct((B, _H, _W), jnp.float32),
        ],
        scratch_shapes=[
            pltpu.VMEM((_H, _W), jnp.int32),
            pltpu.VMEM((_H, _W), jnp.uint32),
        ],
    )(weight, kspace)
    return (
        oks,
        mask,
        jnp.asarray(_RATIO, dtype=jnp.int32),
    )


# speedup vs baseline: 2.2128x; 1.0040x over previous
import jax
import jax.numpy as jnp
from jax import lax
from jax.experimental import pallas as pl
from jax.experimental.pallas import tpu as pltpu

_SLOPE = 5.0
_BUDGET = 1.0 / 16.0 - 1.0 / 128.0
_RATIO = 128
_C_LO = 146
_C_HI = 174
_H = 320
_W = 320
_HW = _H * _W
_BPB = 2

_KS0 = 0
_KS1 = 42
_KS2 = 0x1BD11BDA ^ _KS0 ^ _KS1


def _rotl(x, d):
    return (x << jnp.uint32(d)) | (x >> jnp.uint32(32 - d))


def _threefry2x32_zero_x0(x1_plus_ks1):
    ks0 = jnp.uint32(_KS0)
    ks1 = jnp.uint32(_KS1)
    ks2 = jnp.uint32(_KS2)
    r_a = (13, 15, 26, 6)
    r_b = (17, 29, 16, 24)

    def four_rounds(x0, x1, rots):
        for r in rots:
            x0 = x0 + x1
            x1 = _rotl(x1, r) ^ x0
        return x0, x1

    x0 = x1_plus_ks1
    x1 = _rotl(x1_plus_ks1, 13) ^ x0
    x0, x1 = four_rounds(x0, x1, (15, 26, 6))
    x0 = x0 + ks1
    x1 = x1 + jnp.uint32(_KS2 + 1)
    x0, x1 = four_rounds(x0, x1, r_b)
    x0 = x0 + ks2
    x1 = x1 + jnp.uint32(_KS0 + 2)
    x0, x1 = four_rounds(x0, x1, r_a)
    x0 = x0 + ks0
    x1 = x1 + jnp.uint32(_KS1 + 3)
    x0, x1 = four_rounds(x0, x1, r_b)
    x0 = x0 + ks1
    x1 = x1 + jnp.uint32(_KS2 + 4)
    x0, x1 = four_rounds(x0, x1, r_a)
    x0 = x0 + ks2
    x1 = x1 + jnp.uint32(_KS0 + 5)
    return x0, x1


def _body(w_ref, ks_ref, oks_ref, mask_ref, thr_ref, ju_ref):
    b = pl.program_id(0)

    @pl.when(b == 0)
    def _prep():
        row = lax.broadcasted_iota(jnp.int32, (_H, _W), 0)
        col = lax.broadcasted_iota(jnp.int32, (_H, _W), 1)
        prob = jax.nn.sigmoid(jnp.float32(_SLOPE) * w_ref[...])
        inside = (row >= _C_LO) & (row < _C_HI) & (col >= _C_LO) & (col < _C_HI)
        prob = jnp.where(inside, jnp.float32(0.0), prob)
        xbar = jnp.mean(prob)
        r = jnp.float32(_BUDGET) / xbar
        beta = (jnp.float32(1.0) - jnp.float32(_BUDGET)) / (jnp.float32(1.0) - xbar)
        mr = jnp.where(
            r <= jnp.float32(1.0),
            prob * r,
            jnp.float32(1.0) - (jnp.float32(1.0) - prob) * beta,
        )
        thr_ref[...] = jnp.ceil(mr * jnp.float32(8388608.0)).astype(jnp.int32)
        ju_ref[...] = (row * _W + col).astype(jnp.uint32) + jnp.uint32(_KS1)

    ju = ju_ref[...]
    thr = thr_ref[...]
    for bi in range(_BPB):
        x1 = ju + lax.convert_element_type((b * _BPB + bi) * _HW, jnp.uint32)
        o0, o1 = _threefry2x32_zero_x0(x1)
        mant = lax.convert_element_type((o0 ^ o1) >> jnp.uint32(9), jnp.int32)
        m = (mant < thr).astype(jnp.float32)
        mask_ref[bi] = m
        oks_ref[bi] = ks_ref[bi] * m[None]


@jax.jit
def kernel(kspace, weight):
    B, C = kspace.shape[0], kspace.shape[1]
    oks, mask = pl.pallas_call(
        _body,
        grid=(B // _BPB,),
        in_specs=[
            pl.BlockSpec((_H, _W), lambda b: (0, 0)),
            pl.BlockSpec((_BPB, C, _H, _W), lambda b: (b, 0, 0, 0)),
        ],
        out_specs=[
            pl.BlockSpec((_BPB, C, _H, _W), lambda b: (b, 0, 0, 0)),
            pl.BlockSpec((_BPB, _H, _W), lambda b: (b, 0, 0)),
        ],
        out_shape=[
            jax.ShapeDtypeStruct((B, C, _H, _W), jnp.float32),
            jax.ShapeDtypeStruct((B, _H, _W), jnp.float32),
        ],
        scratch_shapes=[
            pltpu.VMEM((_H, _W), jnp.int32),
            pltpu.VMEM((_H, _W), jnp.uint32),
        ],
    )(weight, kspace)
    return (
        oks,
        mask,
        jnp.asarray(_RATIO, dtype=jnp.int32),
    )
